# baseline (speedup 1.0000x reference)
import functools

import jax
import jax.numpy as jnp
from jax import lax
from jax.experimental import pallas as pl
from jax.experimental.pallas import tpu as pltpu
from jax.experimental.pallas import tpu_sc as plsc

N = 10000
E = 320000
NC = 2
NS = 16
L = 16
NW = NC * NS
EPW = E // NW
CH = 80
NCHUNK = EPW // CH
NPAIR = (NCHUNK - 1) // 2
NG = CH // L
RPT = 624
ZR = 208
TAIL = N - NS * RPT


def _mm_bias(x, W, b, bn):
    n = x.shape[0]
    f = x.shape[1]
    h = W.shape[0]

    def body(x_ref, w_ref, b_ref, o_ref):
        o_ref[...] = lax.dot_general(
            x_ref[...], w_ref[...], (((1,), (1,)), ((), ())),
            preferred_element_type=jnp.float32) + b_ref[...]

    return pl.pallas_call(
        body,
        grid=(n // bn,),
        in_specs=[
            pl.BlockSpec((bn, f), lambda i: (i, 0)),
            pl.BlockSpec((h, f), lambda i: (0, 0)),
            pl.BlockSpec((1, h), lambda i: (0, 0)),
        ],
        out_specs=pl.BlockSpec((bn, h), lambda i: (i, 0)),
        out_shape=jax.ShapeDtypeStruct((n, h), jnp.float32),
    )(x, W, b.reshape(1, h))


def _uv_vectors(mw0r, mw1):
    h = mw1.shape[0]

    def body(a_ref, w_ref, o_ref):
        a = a_ref[...]
        u = jnp.where(a > 0, a, 0.2 * a)
        v = jnp.where(a < 0, a, 0.2 * a)
        uv = jnp.concatenate([u, v], axis=0)
        o_ref[...] = lax.dot_general(
            uv, w_ref[...], (((1,), (1,)), ((), ())),
            preferred_element_type=jnp.float32)

    return pl.pallas_call(
        body,
        out_shape=jax.ShapeDtypeStruct((2, h), jnp.float32),
    )(mw0r, mw1)


def _div_eps(h, d0, d1, bn):
    n, f = h.shape

    def body(h_ref, a_ref, b_ref, o_ref):
        o_ref[...] = h_ref[...] / (a_ref[...] + b_ref[...] + 1e-16)

    spec = pl.BlockSpec((bn, f), lambda i: (i, 0))
    return pl.pallas_call(
        body,
        grid=(n // bn,),
        in_specs=[spec, spec, spec],
        out_specs=spec,
        out_shape=jax.ShapeDtypeStruct((n, f), jnp.float32),
    )(h, d0, d1)


def _elu_mm_bias(o0, o1, W, b, bn):
    n, f = o0.shape
    h = W.shape[0]

    def body(a_ref, b_ref, w_ref, bias_ref, o_ref):
        t = a_ref[...] + b_ref[...]
        t = jnp.where(t > 0, t, jnp.exp(t) - 1.0)
        o_ref[...] = lax.dot_general(
            t, w_ref[...], (((1,), (1,)), ((), ())),
            preferred_element_type=jnp.float32) + bias_ref[...]

    spec = pl.BlockSpec((bn, f), lambda i: (i, 0))
    return pl.pallas_call(
        body,
        grid=(n // bn,),
        in_specs=[
            spec, spec,
            pl.BlockSpec((h, f), lambda i: (0, 0)),
            pl.BlockSpec((1, h), lambda i: (0, 0)),
        ],
        out_specs=pl.BlockSpec((bn, h), lambda i: (i, 0)),
        out_shape=jax.ShapeDtypeStruct((n, h), jnp.float32),
    )(o0, o1, W, b.reshape(1, h))


def _log_softmax(o0, o1, bn):
    n, f = o0.shape

    def body(a_ref, b_ref, o_ref):
        z = a_ref[...] + b_ref[...]
        m = jnp.max(z, axis=1, keepdims=True)
        lse = m + jnp.log(jnp.sum(jnp.exp(z - m), axis=1, keepdims=True))
        o_ref[...] = z - lse

    spec = pl.BlockSpec((bn, f), lambda i: (i, 0))
    return pl.pallas_call(
        body,
        grid=(n // bn,),
        in_specs=[spec, spec],
        out_specs=spec,
        out_shape=jax.ShapeDtypeStruct((n, f), jnp.float32),
    )(o0, o1)


_SC_PARAMS = pltpu.CompilerParams(use_tc_tiling_on_sc=False)


def _zero_acc(zbuf, acc, sid, h):
    nb = h // L

    def zrow(i, _):
        for jb in range(nb):
            zbuf[i, pl.ds(jb * L, L)] = jnp.zeros((L,), jnp.float32)
        return 0

    lax.fori_loop(0, ZR, zrow, 0)
    for k in range(RPT // ZR):
        pltpu.sync_copy(zbuf, acc.at[pl.ds(sid * RPT + k * ZR, ZR)])

    @pl.when(sid == 0)
    def _():
        pltpu.sync_copy(zbuf.at[pl.ds(0, TAIL)], acc.at[pl.ds(NS * RPT, TAIL)])


def _copy_out(acc, out_hbm, cid, sid, h):
    for k in range(RPT // ZR):
        r0 = sid * RPT + k * ZR
        pltpu.sync_copy(acc.at[pl.ds(r0, ZR)], out_hbm.at[cid, pl.ds(r0, ZR)])

    @pl.when(sid == 0)
    def _():
        pltpu.sync_copy(acc.at[pl.ds(NS * RPT, TAIL)],
                        out_hbm.at[cid, pl.ds(NS * RPT, TAIL)])


def _sc_pass(h, with_gather):
    nb = h // L
    mesh = plsc.VectorSubcoreMesh(core_axis_name="c", subcore_axis_name="s")

    scratch = [
        pltpu.VMEM((CH,), jnp.int32),
        pltpu.VMEM((CH,), jnp.int32),
        pltpu.VMEM((CH,), jnp.float32),
        pltpu.VMEM((CH,), jnp.float32),
        pltpu.VMEM((CH,), jnp.int32),
        pltpu.VMEM((CH,), jnp.int32),
        pltpu.VMEM((2, h), jnp.float32),
        pltpu.VMEM((CH, h), jnp.float32),
        pltpu.VMEM((CH, h), jnp.float32),
        pltpu.VMEM((ZR, h), jnp.float32),
        pltpu.VMEM_SHARED((N, h), jnp.float32),
        pltpu.SemaphoreType.DMA,
        pltpu.SemaphoreType.DMA,
        pltpu.SemaphoreType.DMA,
        pltpu.SemaphoreType.DMA,
    ]
    if with_gather:
        scratch += [
            pltpu.VMEM((CH,), jnp.int32),
            pltpu.VMEM((CH,), jnp.int32),
            pltpu.VMEM((CH, h), jnp.float32),
            pltpu.VMEM((CH, h), jnp.float32),
            pltpu.SemaphoreType.DMA,
            pltpu.SemaphoreType.DMA,
        ]

    def body(*refs):
        if with_gather:
            (spk_hbm, dpk_hbm, wpk_hbm, uv_hbm, b_hbm, out_hbm,
             ib0, ib1, wb0, wb1, sc0, sc1, uv_v, rows0, rows1, zbuf, acc,
             lsem0, lsem1, ssem0, ssem1,
             db0, db1, brow0, brow1, gsem0, gsem1) = refs
            dbufs = (db0, db1)
            brows = (brow0, brow1)
            gsems = (gsem0, gsem1)
        else:
            (spk_hbm, wpk_hbm, uv_hbm, out_hbm,
             ib0, ib1, wb0, wb1, sc0, sc1, uv_v, rows0, rows1, zbuf, acc,
             lsem0, lsem1, ssem0, ssem1) = refs
        cid = lax.axis_index("c")
        sid = lax.axis_index("s")
        wid = cid * NS + sid
        g0 = wid * NCHUNK

        _zero_acc(zbuf, acc, sid, h)
        pltpu.sync_copy(uv_hbm, uv_v)
        Us = [uv_v[0, pl.ds(jb * L, L)] for jb in range(nb)]
        Vs = [uv_v[1, pl.ds(jb * L, L)] for jb in range(nb)]
        plsc.subcore_barrier()

        ibufs = (ib0, ib1)
        wbufs = (wb0, wb1)
        scrs = (sc0, sc1)
        lsems = (lsem0, lsem1)
        ssems = (ssem0, ssem1)
        rows = (rows0, rows1)

        def issue_load(g, b):
            pltpu.async_copy(spk_hbm.at[g], ibufs[b], lsems[b])
            pltpu.async_copy(wpk_hbm.at[g], wbufs[b], lsems[b])
            if with_gather:
                pltpu.async_copy(dpk_hbm.at[g], dbufs[b], lsems[b])

        def wait_load(b):
            pltpu.make_async_copy(spk_hbm.at[g0], ibufs[b], lsems[b]).wait()
            pltpu.make_async_copy(wpk_hbm.at[g0], wbufs[b], lsems[b]).wait()
            if with_gather:
                pltpu.make_async_copy(spk_hbm.at[g0], dbufs[b], lsems[b]).wait()

        def issue_gather(b):
            pltpu.async_copy(b_hbm.at[ibufs[b]], brows[b], gsems[b])

        def wait_gather(b):
            pltpu.make_async_copy(b_hbm.at[ibufs[b]], brows[b], gsems[b]).wait()

        def drain_scatter(b):
            pltpu.make_async_copy(rows[b], acc.at[scrs[b]], ssems[b],
                                  ).wait()

        def compute(b):
            wb = wbufs[b]
            rv = rows[b]

            def group(g, _):
                wv = wb[pl.ds(g * L, L)]
                for i in range(L):
                    w = wv[i]
                    pos = w > 0
                    for jb in range(nb):
                        s = jnp.where(pos, Us[jb], Vs[jb])
                        val = jnp.exp(w * s)
                        if with_gather:
                            val = val * brows[b][g * L + i, pl.ds(jb * L, L)]
                        rv[g * L + i, pl.ds(jb * L, L)] = val
                return 0

            lax.fori_loop(0, NG, group, 0)
            idx_src = dbufs[b] if with_gather else ibufs[b]
            for k in range(CH // L):
                scrs[b][pl.ds(k * L, L)] = idx_src[pl.ds(k * L, L)]
            pltpu.async_copy(rows[b], acc.at[scrs[b]], ssems[b], add=True)

        issue_load(g0, 0)
        wait_load(0)
        issue_load(g0 + 1, 1)
        if with_gather:
            issue_gather(0)

        def half(b, cur):
            wait_load(1 - b)
            if with_gather:
                wait_gather(b)

            @pl.when(cur >= 2)
            def _():
                drain_scatter(b)

            compute(b)
            if with_gather:
                issue_gather(1 - b)

            @pl.when(cur <= NCHUNK - 3)
            def _():
                issue_load(g0 + cur + 2, b)

        def pair(c2, _):
            half(0, 2 * c2)
            half(1, 2 * c2 + 1)
            return 0

        lax.fori_loop(0, NPAIR, pair, 0)

        if with_gather:
            wait_gather(0)
        drain_scatter(0)
        compute(0)
        drain_scatter(1)
        drain_scatter(0)

        plsc.subcore_barrier()
        _copy_out(acc, out_hbm, cid, sid, h)

    return pl.kernel(
        body,
        out_type=jax.ShapeDtypeStruct((NC, N, h), jnp.float32),
        mesh=mesh,
        compiler_params=_SC_PARAMS,
        scratch_types=scratch,
    )


_SC_DENOM = {h: _sc_pass(h, False) for h in (64, 16)}
_SC_AGG = {h: _sc_pass(h, True) for h in (64, 16)}


def _layer(h_nodes, spk, dpk, wpk, mw0, mw1, hdim):
    uv = _uv_vectors(mw0.reshape(1, hdim), mw1)
    d = _SC_DENOM[hdim](spk, wpk, uv)
    b = _div_eps(h_nodes, d[0], d[1], 1000)
    o = _SC_AGG[hdim](spk, dpk, wpk, uv, b)
    return o[0], o[1]


def kernel(x, edge_index, w_mul, W1, b1, m1w0, m1w1, m1b1,
           W2, b2, m2w0, m2w1, m2b1):
    spk = edge_index[0].reshape(E // CH, CH)
    dpk = edge_index[1].reshape(E // CH, CH)
    wpk = w_mul[:, 0].reshape(E // CH, CH)

    h1 = _mm_bias(x, W1, b1, 1000)
    o1a, o1b = _layer(h1, spk, dpk, wpk, m1w0, m1w1, 64)
    h2 = _elu_mm_bias(o1a, o1b, W2, b2, 1000)
    o2a, o2b = _layer(h2, spk, dpk, wpk, m2w0, m2w1, 16)
    return _log_softmax(o2a, o2b, 1000)


# speedup vs baseline: 1.0995x; 1.0995x over previous
import functools

import jax
import jax.numpy as jnp
from jax import lax
from jax.experimental import pallas as pl
from jax.experimental.pallas import tpu as pltpu
from jax.experimental.pallas import tpu_sc as plsc

N = 10000
E = 320000
NC = 2
NS = 16
L = 16
NW = NC * NS
EPW = E // NW
CH = 80
NCHUNK = EPW // CH
NPAIR = (NCHUNK - 1) // 2
NG = CH // L
RPT = 624
ZR = 208
TAIL = N - NS * RPT


def _mm_bias(x, W, b, bn):
    n = x.shape[0]
    f = x.shape[1]
    h = W.shape[0]

    def body(x_ref, w_ref, b_ref, o_ref):
        o_ref[...] = lax.dot_general(
            x_ref[...], w_ref[...], (((1,), (1,)), ((), ())),
            preferred_element_type=jnp.float32) + b_ref[...]

    return pl.pallas_call(
        body,
        grid=(n // bn,),
        in_specs=[
            pl.BlockSpec((bn, f), lambda i: (i, 0)),
            pl.BlockSpec((h, f), lambda i: (0, 0)),
            pl.BlockSpec((1, h), lambda i: (0, 0)),
        ],
        out_specs=pl.BlockSpec((bn, h), lambda i: (i, 0)),
        out_shape=jax.ShapeDtypeStruct((n, h), jnp.float32),
    )(x, W, b.reshape(1, h))


def _uv_vectors(mw0r, mw1):
    h = mw1.shape[0]

    def body(a_ref, w_ref, o_ref):
        a = a_ref[...]
        u = jnp.where(a > 0, a, 0.2 * a)
        v = jnp.where(a < 0, a, 0.2 * a)
        uv = jnp.concatenate([u, v], axis=0)
        o_ref[...] = lax.dot_general(
            uv, w_ref[...], (((1,), (1,)), ((), ())),
            preferred_element_type=jnp.float32)

    return pl.pallas_call(
        body,
        out_shape=jax.ShapeDtypeStruct((2, h), jnp.float32),
    )(mw0r, mw1)


def _div_eps(h, d0, d1, bn):
    n, f = h.shape

    def body(h_ref, a_ref, b_ref, o_ref):
        o_ref[...] = h_ref[...] / (a_ref[...] + b_ref[...] + 1e-16)

    spec = pl.BlockSpec((bn, f), lambda i: (i, 0))
    return pl.pallas_call(
        body,
        grid=(n // bn,),
        in_specs=[spec, spec, spec],
        out_specs=spec,
        out_shape=jax.ShapeDtypeStruct((n, f), jnp.float32),
    )(h, d0, d1)


def _elu_mm_bias(o0, o1, W, b, bn):
    n, f = o0.shape
    h = W.shape[0]

    def body(a_ref, b_ref, w_ref, bias_ref, o_ref):
        t = a_ref[...] + b_ref[...]
        t = jnp.where(t > 0, t, jnp.exp(t) - 1.0)
        o_ref[...] = lax.dot_general(
            t, w_ref[...], (((1,), (1,)), ((), ())),
            preferred_element_type=jnp.float32) + bias_ref[...]

    spec = pl.BlockSpec((bn, f), lambda i: (i, 0))
    return pl.pallas_call(
        body,
        grid=(n // bn,),
        in_specs=[
            spec, spec,
            pl.BlockSpec((h, f), lambda i: (0, 0)),
            pl.BlockSpec((1, h), lambda i: (0, 0)),
        ],
        out_specs=pl.BlockSpec((bn, h), lambda i: (i, 0)),
        out_shape=jax.ShapeDtypeStruct((n, h), jnp.float32),
    )(o0, o1, W, b.reshape(1, h))


def _log_softmax(o0, o1, bn):
    n, f = o0.shape

    def body(a_ref, b_ref, o_ref):
        z = a_ref[...] + b_ref[...]
        m = jnp.max(z, axis=1, keepdims=True)
        lse = m + jnp.log(jnp.sum(jnp.exp(z - m), axis=1, keepdims=True))
        o_ref[...] = z - lse

    spec = pl.BlockSpec((bn, f), lambda i: (i, 0))
    return pl.pallas_call(
        body,
        grid=(n // bn,),
        in_specs=[spec, spec],
        out_specs=spec,
        out_shape=jax.ShapeDtypeStruct((n, f), jnp.float32),
    )(o0, o1)


_SC_PARAMS = pltpu.CompilerParams(use_tc_tiling_on_sc=False)


def _zero_acc(zbuf, acc, sid, h):
    nb = h // L

    def zrow(i, _):
        for jb in range(nb):
            zbuf[i, pl.ds(jb * L, L)] = jnp.zeros((L,), jnp.float32)
        return 0

    lax.fori_loop(0, ZR, zrow, 0)
    for k in range(RPT // ZR):
        pltpu.sync_copy(zbuf, acc.at[pl.ds(sid * RPT + k * ZR, ZR)])

    @pl.when(sid == 0)
    def _():
        pltpu.sync_copy(zbuf.at[pl.ds(0, TAIL)], acc.at[pl.ds(NS * RPT, TAIL)])


def _copy_out(acc, out_hbm, cid, sid, h):
    for k in range(RPT // ZR):
        r0 = sid * RPT + k * ZR
        pltpu.sync_copy(acc.at[pl.ds(r0, ZR)], out_hbm.at[cid, pl.ds(r0, ZR)])

    @pl.when(sid == 0)
    def _():
        pltpu.sync_copy(acc.at[pl.ds(NS * RPT, TAIL)],
                        out_hbm.at[cid, pl.ds(NS * RPT, TAIL)])


def _sc_pass(h, with_gather):
    nb = h // L
    mesh = plsc.VectorSubcoreMesh(core_axis_name="c", subcore_axis_name="s")

    scratch = [
        pltpu.VMEM((CH,), jnp.int32),
        pltpu.VMEM((CH,), jnp.int32),
        pltpu.VMEM((CH,), jnp.float32),
        pltpu.VMEM((CH,), jnp.float32),
        pltpu.VMEM((CH,), jnp.int32),
        pltpu.VMEM((CH,), jnp.int32),
        pltpu.VMEM((2, h), jnp.float32),
        pltpu.VMEM((CH, h), jnp.float32),
        pltpu.VMEM((CH, h), jnp.float32),
        pltpu.VMEM((ZR, h), jnp.float32),
        pltpu.VMEM_SHARED((N, h), jnp.float32),
        pltpu.SemaphoreType.DMA,
        pltpu.SemaphoreType.DMA,
        pltpu.SemaphoreType.DMA,
        pltpu.SemaphoreType.DMA,
    ]
    if with_gather:
        scratch += [
            pltpu.VMEM((CH,), jnp.int32),
            pltpu.VMEM((CH,), jnp.int32),
            pltpu.VMEM((CH, h), jnp.float32),
            pltpu.VMEM((CH, h), jnp.float32),
            pltpu.SemaphoreType.DMA,
            pltpu.SemaphoreType.DMA,
        ]

    def body(*refs):
        if with_gather:
            (spk_hbm, dpk_hbm, wpk_hbm, uv_hbm, b_hbm, out_hbm,
             ib0, ib1, wb0, wb1, sc0, sc1, uv_v, rows0, rows1, zbuf, acc,
             lsem0, lsem1, ssem0, ssem1,
             db0, db1, brow0, brow1, gsem0, gsem1) = refs
            dbufs = (db0, db1)
            brows = (brow0, brow1)
            gsems = (gsem0, gsem1)
        else:
            (spk_hbm, wpk_hbm, uv_hbm, out_hbm,
             ib0, ib1, wb0, wb1, sc0, sc1, uv_v, rows0, rows1, zbuf, acc,
             lsem0, lsem1, ssem0, ssem1) = refs
        cid = lax.axis_index("c")
        sid = lax.axis_index("s")
        wid = cid * NS + sid
        g0 = wid * NCHUNK

        _zero_acc(zbuf, acc, sid, h)
        pltpu.sync_copy(uv_hbm, uv_v)
        Us = [uv_v[0, pl.ds(jb * L, L)] for jb in range(nb)]
        Vs = [uv_v[1, pl.ds(jb * L, L)] for jb in range(nb)]
        plsc.subcore_barrier()

        ibufs = (ib0, ib1)
        wbufs = (wb0, wb1)
        scrs = (sc0, sc1)
        lsems = (lsem0, lsem1)
        ssems = (ssem0, ssem1)
        rows = (rows0, rows1)

        def issue_load(g, b):
            pltpu.async_copy(spk_hbm.at[g], ibufs[b], lsems[b])
            pltpu.async_copy(wpk_hbm.at[g], wbufs[b], lsems[b])
            if with_gather:
                pltpu.async_copy(dpk_hbm.at[g], dbufs[b], lsems[b])

        def wait_load(b):
            pltpu.make_async_copy(spk_hbm.at[g0], ibufs[b], lsems[b]).wait()
            pltpu.make_async_copy(wpk_hbm.at[g0], wbufs[b], lsems[b]).wait()
            if with_gather:
                pltpu.make_async_copy(spk_hbm.at[g0], dbufs[b], lsems[b]).wait()

        def issue_gather(b):
            pltpu.async_copy(b_hbm.at[ibufs[b]], brows[b], gsems[b])

        def wait_gather(b):
            pltpu.make_async_copy(b_hbm.at[ibufs[b]], brows[b], gsems[b]).wait()

        def drain_scatter(b):
            pltpu.make_async_copy(rows[b], acc.at[scrs[b]], ssems[b],
                                  ).wait()

        def compute(b):
            wb = wbufs[b]
            rv = rows[b]

            def group(g, _):
                wv = wb[pl.ds(g * L, L)]
                for i in range(L):
                    w = wv[i]
                    pos = w > 0
                    for jb in range(nb):
                        s = jnp.where(pos, Us[jb], Vs[jb])
                        val = jnp.exp(w * s)
                        if with_gather:
                            val = val * brows[b][g * L + i, pl.ds(jb * L, L)]
                        rv[g * L + i, pl.ds(jb * L, L)] = val
                return 0

            lax.fori_loop(0, NG, group, 0)
            idx_src = dbufs[b] if with_gather else ibufs[b]
            for k in range(CH // L):
                scrs[b][pl.ds(k * L, L)] = idx_src[pl.ds(k * L, L)]
            pltpu.async_copy(rows[b], acc.at[scrs[b]], ssems[b], add=True)

        issue_load(g0, 0)
        wait_load(0)
        issue_load(g0 + 1, 1)
        if with_gather:
            issue_gather(0)

        def half(b, cur):
            wait_load(1 - b)
            if with_gather:
                wait_gather(b)

            @pl.when(cur >= 2)
            def _():
                drain_scatter(b)

            compute(b)
            if with_gather:
                issue_gather(1 - b)

            @pl.when(cur <= NCHUNK - 3)
            def _():
                issue_load(g0 + cur + 2, b)

        def pair(c2, _):
            half(0, 2 * c2)
            half(1, 2 * c2 + 1)
            return 0

        lax.fori_loop(0, NPAIR, pair, 0)

        if with_gather:
            wait_gather(0)
        drain_scatter(0)
        compute(0)
        drain_scatter(1)
        drain_scatter(0)

        plsc.subcore_barrier()
        _copy_out(acc, out_hbm, cid, sid, h)

    return pl.kernel(
        body,
        out_type=jax.ShapeDtypeStruct((NC, N, h), jnp.float32),
        mesh=mesh,
        compiler_params=_SC_PARAMS,
        scratch_types=scratch,
    )


def _sc_denom2():
    mesh = plsc.VectorSubcoreMesh(core_axis_name="c", subcore_axis_name="s")

    scratch = [
        pltpu.VMEM((CH,), jnp.int32),
        pltpu.VMEM((CH,), jnp.int32),
        pltpu.VMEM((CH,), jnp.float32),
        pltpu.VMEM((CH,), jnp.float32),
        pltpu.VMEM((2, 64), jnp.float32),
        pltpu.VMEM((2, 16), jnp.float32),
        pltpu.VMEM((CH, 64), jnp.float32),
        pltpu.VMEM((CH, 16), jnp.float32),
        pltpu.VMEM((ZR, 64), jnp.float32),
        pltpu.VMEM_SHARED((N, 64), jnp.float32),
        pltpu.VMEM_SHARED((N, 16), jnp.float32),
        pltpu.SemaphoreType.DMA,
        pltpu.SemaphoreType.DMA,
    ]

    def body(spk_hbm, wpk_hbm, uva_hbm, uvb_hbm, outa_hbm, outb_hbm,
             ib0, ib1, wb0, wb1, uva_v, uvb_v, rowsa, rowsb, zbuf,
             acca, accb, sem0, sem1):
        cid = lax.axis_index("c")
        sid = lax.axis_index("s")
        wid = cid * NS + sid
        g0 = wid * NCHUNK

        _zero_acc(zbuf, acca, sid, 64)
        for k in range(RPT // ZR):
            pltpu.sync_copy(zbuf.at[pl.ds(0, ZR), pl.ds(0, 16)],
                            accb.at[pl.ds(sid * RPT + k * ZR, ZR)])

        @pl.when(sid == 0)
        def _():
            pltpu.sync_copy(zbuf.at[pl.ds(0, TAIL), pl.ds(0, 16)],
                            accb.at[pl.ds(NS * RPT, TAIL)])

        pltpu.sync_copy(uva_hbm, uva_v)
        pltpu.sync_copy(uvb_hbm, uvb_v)
        Ua = [uva_v[0, pl.ds(jb * L, L)] for jb in range(4)]
        Va = [uva_v[1, pl.ds(jb * L, L)] for jb in range(4)]
        Ub = uvb_v[0, pl.ds(0, L)]
        Vb = uvb_v[1, pl.ds(0, L)]
        plsc.subcore_barrier()

        ibufs = (ib0, ib1)
        wbufs = (wb0, wb1)
        sems = (sem0, sem1)

        def compute_and_scatter(b):
            ib, wb = ibufs[b], wbufs[b]

            def group(g, _):
                wv = wb[pl.ds(g * L, L)]
                for i in range(L):
                    w = wv[i]
                    pos = w > 0
                    for jb in range(4):
                        s = jnp.where(pos, Ua[jb], Va[jb])
                        rowsa[g * L + i, pl.ds(jb * L, L)] = jnp.exp(w * s)
                    sb = jnp.where(pos, Ub, Vb)
                    rowsb[g * L + i, pl.ds(0, L)] = jnp.exp(w * sb)
                return 0

            lax.fori_loop(0, NG, group, 0)
            pltpu.sync_copy(rowsa, acca.at[ib], add=True)
            pltpu.sync_copy(rowsb, accb.at[ib], add=True)

        def start_load(g, b):
            pltpu.async_copy(spk_hbm.at[g], ibufs[b], sems[b])
            pltpu.async_copy(wpk_hbm.at[g], wbufs[b], sems[b])

        def wait_load(b):
            pltpu.make_async_copy(spk_hbm.at[g0], ibufs[b], sems[b]).wait()
            pltpu.make_async_copy(wpk_hbm.at[g0], wbufs[b], sems[b]).wait()

        start_load(g0, 0)

        def pair(c2, _):
            for b in range(2):
                cur = 2 * c2 + b
                wait_load(b)
                start_load(g0 + cur + 1, 1 - b)
                compute_and_scatter(b)
            return 0

        lax.fori_loop(0, NPAIR, pair, 0)
        wait_load(0)
        compute_and_scatter(0)

        plsc.subcore_barrier()
        _copy_out(acca, outa_hbm, cid, sid, 64)
        for k in range(RPT // ZR):
            r0 = sid * RPT + k * ZR
            pltpu.sync_copy(accb.at[pl.ds(r0, ZR)], outb_hbm.at[cid, pl.ds(r0, ZR)])

        @pl.when(sid == 0)
        def _():
            pltpu.sync_copy(accb.at[pl.ds(NS * RPT, TAIL)],
                            outb_hbm.at[cid, pl.ds(NS * RPT, TAIL)])

    return pl.kernel(
        body,
        out_type=(jax.ShapeDtypeStruct((NC, N, 64), jnp.float32),
                  jax.ShapeDtypeStruct((NC, N, 16), jnp.float32)),
        mesh=mesh,
        compiler_params=_SC_PARAMS,
        scratch_types=scratch,
    )


_SC_DENOM2 = _sc_denom2()

_SC_AGG = {h: _sc_pass(h, True) for h in (64, 16)}


def _half_layer(h_nodes, spk, dpk, wpk, uv, d, hdim):
    b = _div_eps(h_nodes, d[0], d[1], 1000)
    o = _SC_AGG[hdim](spk, dpk, wpk, uv, b)
    return o[0], o[1]


def kernel(x, edge_index, w_mul, W1, b1, m1w0, m1w1, m1b1,
           W2, b2, m2w0, m2w1, m2b1):
    spk = edge_index[0].reshape(E // CH, CH)
    dpk = edge_index[1].reshape(E // CH, CH)
    wpk = w_mul[:, 0].reshape(E // CH, CH)

    uv1 = _uv_vectors(m1w0.reshape(1, 64), m1w1)
    uv2 = _uv_vectors(m2w0.reshape(1, 16), m2w1)
    d1, d2 = _SC_DENOM2(spk, wpk, uv1, uv2)

    h1 = _mm_bias(x, W1, b1, 1000)
    o1a, o1b = _half_layer(h1, spk, dpk, wpk, uv1, d1, 64)
    h2 = _elu_mm_bias(o1a, o1b, W2, b2, 1000)
    o2a, o2b = _half_layer(h2, spk, dpk, wpk, uv2, d2, 16)
    return _log_softmax(o2a, o2b, 1000)
